# baseline (speedup 1.0000x reference)
import jax
import jax.numpy as jnp
from jax.experimental import pallas as pl

K = 1024
BLK = 1024


def _mv_mxu(x_ref, v_ref, y_ref):
    y_ref[...] = jnp.dot(x_ref[...], v_ref[...],
                         preferred_element_type=jnp.float32)[:, 0]


def _mv_vpu(x_ref, v_ref, y_ref):
    y_ref[...] = jnp.sum(x_ref[...] * v_ref[...][:, 0][None, :], axis=1)


def _matvec(x, v, body):
    n = x.shape[0]
    grid = (n + BLK - 1) // BLK
    return pl.pallas_call(
        body,
        grid=(grid,),
        in_specs=[
            pl.BlockSpec((BLK, 128), lambda i: (i, 0)),
            pl.BlockSpec((128, 1), lambda i: (0, 0)),
        ],
        out_specs=pl.BlockSpec((BLK,), lambda i: (i,)),
        out_shape=jax.ShapeDtypeStruct((n,), jnp.float32),
    )(x, v)


def kernel(x, learnable_vector):
    length = jnp.linalg.norm(learnable_vector)
    y_mxu = _matvec(x, learnable_vector, _mv_mxu) / length
    vals, idx = jax.lax.top_k(y_mxu, K)
    x_part = jnp.take(x, idx, axis=0)
    return jnp.multiply(x_part, vals[:, None])


# speedup vs baseline: 1.2154x; 1.2154x over previous
import functools

import jax
import jax.numpy as jnp
from jax import lax
from jax.experimental import pallas as pl
from jax.experimental.pallas import tpu as pltpu
from jax.experimental.pallas import tpu_sc as plsc

TOPK = 1024
N = 100000
D = 128
NS = 16
BLK_A = 1024
NPAD = 98 * BLK_A
CHUNK = NPAD // NS
NV = CHUNK // 16
GT_CAP = 1040
EQ_CAP = 96
CAND = 2048
SCAND = CAND + 128
SLOTS = CAND // NS
OUT_PAD = TOPK + NS


def _keys_body(x_ref, v_ref, len_ref, k_ref):
    i = pl.program_id(0)
    y = jnp.dot(x_ref[...], v_ref[...],
                preferred_element_type=jnp.float32)[:, 0]
    y = y / len_ref[0, 0]
    u = lax.bitcast_convert_type(y, jnp.uint32)
    top = (u >> 31).astype(jnp.bool_)
    key = jnp.where(top, ~u, u | jnp.uint32(0x80000000))
    row = i * BLK_A + lax.iota(jnp.int32, BLK_A)
    k_ref[...] = jnp.where(row < N, key, jnp.uint32(0))


def _make_keys(x, v, length):
    return pl.pallas_call(
        _keys_body,
        grid=(NPAD // BLK_A,),
        in_specs=[
            pl.BlockSpec((BLK_A, D), lambda i: (i, 0)),
            pl.BlockSpec((D, 1), lambda i: (0, 0)),
            pl.BlockSpec((1, 1), lambda i: (0, 0)),
        ],
        out_specs=pl.BlockSpec((BLK_A,), lambda i: (i,)),
        out_shape=jax.ShapeDtypeStruct((NPAD,), jnp.uint32),
    )(x, v, length)


def _sc_body(keys_hbm, x_hbm, out_hbm,
             keys_v, hist_v, bins_v, gbins_v,
             gtk_v, gti_v, eqi_v, tk_v, posg_v, pose_v,
             cnt_v, allcnt_v, pad_k_v, pad_i_v,
             ck_v, ci_v, selk_v, rank_v, gidx_v, vals_v, rows_v,
             shist, scnt, scandk, scandi, sem):
    s = lax.axis_index("s")
    base = s * CHUNK
    lane = lax.iota(jnp.int32, 16)
    zeros16 = jnp.zeros((16,), jnp.int32)
    ones16 = jnp.ones((16,), jnp.int32)

    pltpu.sync_copy(keys_hbm.at[pl.ds(base, CHUNK)], keys_v)

    prefix = jnp.uint32(0)
    gt_total = jnp.int32(0)
    for r in range(4):
        sh_dig = 24 - 8 * r

        def zero_step(i, c):
            hist_v[pl.ds(i * 16, 16)] = zeros16
            return c
        lax.fori_loop(0, 256, zero_step, 0)

        def hist_step(i, c):
            kv = keys_v[pl.ds(i * 16, 16)]
            dig = ((kv >> sh_dig) & jnp.uint32(0xFF)).astype(jnp.int32)
            slot = lane * 256 + dig
            if r == 0:
                m = jnp.full((16,), True)
            else:
                m = (kv >> (32 - 8 * r)) == prefix
            plsc.addupdate_scatter(hist_v, [slot], ones16, mask=m)
            return c
        lax.fori_loop(0, NV, hist_step, 0)

        def lred_step(c, _):
            acc = zeros16
            for l in range(NS):
                acc = acc + hist_v[pl.ds(l * 256 + c * 16, 16)]
            bins_v[pl.ds(c * 16, 16)] = acc
            return _
        lax.fori_loop(0, 16, lred_step, 0)

        pltpu.sync_copy(bins_v.at[pl.ds(0, 256)],
                        shist.at[pl.ds(s * 256, 256)])
        plsc.subcore_barrier()
        pltpu.sync_copy(shist, gbins_v)
        plsc.subcore_barrier()

        def gred_step(c, _):
            acc = zeros16
            for w in range(NS):
                acc = acc + gbins_v[pl.ds(w * 256 + c * 16, 16)]
            bins_v[pl.ds(c * 16, 16)] = acc
            return _
        lax.fori_loop(0, 16, gred_step, 0)

        k_rem = TOPK - gt_total

        def scan_step(t, carry):
            cum, bstar, gt_above = carry
            b = 255 - t
            cnt = bins_v[pl.ds(b, 16)][0]
            newcum = cum + cnt
            hit = (newcum >= k_rem) & (bstar < 0)
            return (newcum,
                    jnp.where(hit, b, bstar),
                    jnp.where(hit, cum, gt_above))
        _, bstar, gt_above = lax.fori_loop(
            0, 256, scan_step,
            (jnp.int32(0), jnp.int32(-1), jnp.int32(0)))
        gt_total = gt_total + gt_above
        prefix = (prefix << 8) | bstar.astype(jnp.uint32)

    thr = prefix
    n_gt = gt_total

    def comp_step(i, carry):
        cg, ce = carry
        kv = keys_v[pl.ds(i * 16, 16)]
        idxv = base + i * 16 + lane
        mgt = kv > thr
        meq = kv == thr
        plsc.store_compressed(gtk_v.at[pl.ds(cg, 16)], kv, mask=mgt)
        plsc.store_compressed(gti_v.at[pl.ds(cg, 16)], idxv, mask=mgt)
        ng = jnp.sum(mgt.astype(jnp.int32))
        ne = jnp.sum(meq.astype(jnp.int32))
        ok_eq = ce <= EQ_CAP - 16

        @pl.when(ok_eq)
        def _():
            plsc.store_compressed(eqi_v.at[pl.ds(ce, 16)], idxv, mask=meq)
        return (cg + ng, jnp.where(ok_eq, ce + ne, ce))
    cnt_gt, cnt_eq = lax.fori_loop(0, NV, comp_step,
                                   (jnp.int32(0), jnp.int32(0)))

    cnt_v[...] = jnp.where(lane == 0, cnt_gt,
                           jnp.where(lane == 1, cnt_eq, 0))
    pltpu.sync_copy(cnt_v, scnt.at[pl.ds(s * 16, 16)])
    plsc.subcore_barrier()
    pltpu.sync_copy(scnt, allcnt_v)
    plsc.subcore_barrier()

    def off_step(w, carry):
        go, eo, gtot, etot = carry
        row = allcnt_v[pl.ds(w * 16, 16)]
        cg = row[0]
        ce = row[1]
        before = w < s
        return (go + jnp.where(before, cg, 0),
                eo + jnp.where(before, ce, 0),
                gtot + cg, etot + ce)
    gt_off, eq_off, n_gt2, eq_tot = lax.fori_loop(
        0, NS, off_step,
        (jnp.int32(0), jnp.int32(0), jnp.int32(0), jnp.int32(0)))

    @pl.when(s == 0)
    def _():
        def pad_step(i, c):
            pad_k_v[pl.ds(i * 16, 16)] = jnp.zeros((16,), jnp.uint32)
            pad_i_v[pl.ds(i * 16, 16)] = jnp.full((16,), 0x7FFFFFFF,
                                                  jnp.int32)
            return c
        lax.fori_loop(0, SCAND // 16, pad_step, 0)
        pltpu.sync_copy(pad_k_v, scandk)
        pltpu.sync_copy(pad_i_v, scandi)
    plsc.subcore_barrier()

    dump = CAND + s * 8

    def posg_step(j, c):
        val = gt_off + j * 16 + lane
        ok = (j * 16 + lane) < cnt_gt
        posg_v[pl.ds(j * 16, 16)] = jnp.where(ok, val, dump)
        return c
    lax.fori_loop(0, GT_CAP // 16, posg_step, 0)

    def pose_step(j, c):
        val = TOPK + eq_off + j * 16 + lane
        ok = (j * 16 + lane) < cnt_eq
        pose_v[pl.ds(j * 16, 16)] = jnp.where(ok, val, dump)
        tk_v[pl.ds(j * 16, 16)] = jnp.full((16,), thr, jnp.uint32)
        return c
    lax.fori_loop(0, EQ_CAP // 16, pose_step, 0)

    pltpu.sync_copy(gtk_v, scandk.at[posg_v])
    pltpu.sync_copy(gti_v, scandi.at[posg_v])
    pltpu.sync_copy(tk_v, scandk.at[pose_v])
    pltpu.sync_copy(eqi_v, scandi.at[pose_v])
    plsc.subcore_barrier()

    pltpu.sync_copy(scandk.at[pl.ds(0, CAND)], ck_v.at[pl.ds(0, CAND)])
    pltpu.sync_copy(scandi.at[pl.ds(0, CAND)], ci_v.at[pl.ds(0, CAND)])

    nv_gt = (n_gt2 + 15) // 16
    nv_eq = (eq_tot + 15) // 16

    def slot_step(t, c):
        j = s + t * NS
        kj = ck_v[pl.ds(j, 16)][0]
        ij = ci_v[pl.ds(j, 16)][0]

        def beat_count(v, acc):
            kv = ck_v[pl.ds(v * 16, 16)]
            iv = ci_v[pl.ds(v * 16, 16)]
            b = (kv > kj) | ((kv == kj) & (iv < ij))
            return acc + b.astype(jnp.int32)
        acc = lax.fori_loop(0, nv_gt, beat_count, zeros16)
        acc = lax.fori_loop(64, 64 + nv_eq, beat_count, acc)
        rank = jnp.sum(acc)
        real = (j < n_gt2) | ((j >= TOPK) & (j < TOPK + eq_tot))
        tvec = jnp.full((16,), t, jnp.int32)
        m0 = lane == 0
        rk = jnp.where(real & (rank < TOPK), rank, TOPK + s)
        plsc.store_scatter(rank_v, [tvec], jnp.full((16,), rk, jnp.int32),
                           mask=m0)
        plsc.store_scatter(gidx_v, [tvec],
                           jnp.full((16,), jnp.where(real, ij, 0), jnp.int32),
                           mask=m0)
        kj_i = lax.bitcast_convert_type(kj, jnp.int32)
        plsc.store_scatter(selk_v, [tvec], jnp.full((16,), kj_i, jnp.int32),
                           mask=m0)
        return c
    lax.fori_loop(0, SLOTS, slot_step, 0)

    def val_step(t, c):
        u = lax.bitcast_convert_type(selk_v[pl.ds(t * 16, 16)], jnp.uint32)
        top = (u >> 31) == jnp.uint32(1)
        bits = jnp.where(top, u ^ jnp.uint32(0x80000000), ~u)
        vals_v[pl.ds(t * 16, 16)] = lax.bitcast_convert_type(bits,
                                                             jnp.float32)
        return c
    lax.fori_loop(0, SLOTS // 16, val_step, 0)

    pltpu.async_copy(x_hbm.at[gidx_v], rows_v, sem).wait()

    def scale_step(t, c):
        v = vals_v[pl.ds(t, 16)][0]
        for seg in range(D // 16):
            sl = pl.ds(seg * 16, 16)
            rows_v[t, sl] = rows_v[t, sl] * v
        return c
    lax.fori_loop(0, SLOTS, scale_step, 0)

    pltpu.sync_copy(rows_v, out_hbm.at[rank_v])


@jax.jit
def _pipeline(x, v):
    length = jnp.linalg.norm(v)
    keys = _make_keys(x, v, length.reshape(1, 1))

    mesh = plsc.VectorSubcoreMesh(core_axis_name="c", subcore_axis_name="s",
                                  num_cores=1)
    sc = pl.kernel(
        _sc_body,
        out_type=jax.ShapeDtypeStruct((OUT_PAD, D), jnp.float32),
        mesh=mesh,
        compiler_params=pltpu.CompilerParams(needs_layout_passes=False),
        scratch_types=[
            pltpu.VMEM((CHUNK,), jnp.uint32),
            pltpu.VMEM((4096,), jnp.int32),
            pltpu.VMEM((256 + 16,), jnp.int32),
            pltpu.VMEM((NS * 256,), jnp.int32),
            pltpu.VMEM((GT_CAP,), jnp.uint32),
            pltpu.VMEM((GT_CAP,), jnp.int32),
            pltpu.VMEM((EQ_CAP,), jnp.int32),
            pltpu.VMEM((EQ_CAP,), jnp.uint32),
            pltpu.VMEM((GT_CAP,), jnp.int32),
            pltpu.VMEM((EQ_CAP,), jnp.int32),
            pltpu.VMEM((16,), jnp.int32),
            pltpu.VMEM((NS * 16,), jnp.int32),
            pltpu.VMEM((SCAND,), jnp.uint32),
            pltpu.VMEM((SCAND,), jnp.int32),
            pltpu.VMEM((CAND + 16,), jnp.uint32),
            pltpu.VMEM((CAND + 16,), jnp.int32),
            pltpu.VMEM((SLOTS,), jnp.int32),
            pltpu.VMEM((SLOTS,), jnp.int32),
            pltpu.VMEM((SLOTS,), jnp.int32),
            pltpu.VMEM((SLOTS + 16,), jnp.float32),
            pltpu.VMEM((SLOTS, D), jnp.float32),
            pltpu.VMEM_SHARED((NS * 256,), jnp.int32),
            pltpu.VMEM_SHARED((NS * 16,), jnp.int32),
            pltpu.VMEM_SHARED((SCAND,), jnp.uint32),
            pltpu.VMEM_SHARED((SCAND,), jnp.int32),
            pltpu.SemaphoreType.DMA,
        ],
    )
    out_pad = sc(keys, x)
    return out_pad[:TOPK]


def kernel(x, learnable_vector):
    return _pipeline(x, learnable_vector)


# speedup vs baseline: 1.6867x; 1.3878x over previous
import functools

import jax
import jax.numpy as jnp
from jax import lax
from jax.experimental import pallas as pl
from jax.experimental.pallas import tpu as pltpu
from jax.experimental.pallas import tpu_sc as plsc

TOPK = 1024
N = 100000
D = 128
NS = 16
BLK_A = 1024
NPAD = 98 * BLK_A
CHUNK = NPAD // NS
NV = CHUNK // 16
GT_CAP = 1040
EQ_CAP = 96
CAND = 2048
SCAND = CAND + 128
SLOTS = CAND // NS
OUT_PAD = TOPK + NS


def _keys_body(x_ref, v_ref, len_ref, k_ref):
    i = pl.program_id(0)
    y = jnp.dot(x_ref[...], v_ref[...],
                preferred_element_type=jnp.float32)[:, 0]
    y = y / len_ref[0, 0]
    u = lax.bitcast_convert_type(y, jnp.uint32)
    top = (u >> 31).astype(jnp.bool_)
    key = jnp.where(top, ~u, u | jnp.uint32(0x80000000))
    row = i * BLK_A + lax.iota(jnp.int32, BLK_A)
    k_ref[...] = jnp.where(row < N, key, jnp.uint32(0))


def _make_keys(x, v, length):
    return pl.pallas_call(
        _keys_body,
        grid=(NPAD // BLK_A,),
        in_specs=[
            pl.BlockSpec((BLK_A, D), lambda i: (i, 0)),
            pl.BlockSpec((D, 1), lambda i: (0, 0)),
            pl.BlockSpec((1, 1), lambda i: (0, 0)),
        ],
        out_specs=pl.BlockSpec((BLK_A,), lambda i: (i,)),
        out_shape=jax.ShapeDtypeStruct((NPAD,), jnp.uint32),
    )(x, v, length)


def _sc_body(keys_hbm, x_hbm, out_hbm,
             keys_v, hist_v, bins_v, gbins_v,
             gtk_v, gti_v, eqi_v, tk_v, posg_v, pose_v,
             cnt_v, allcnt_v, pad_k_v, pad_i_v,
             ck_v, ci_v, selk_v, rank_v, gidx_v, vals_v, rows_v,
             shist, scnt, scandk, scandi, sem):
    s = lax.axis_index("s")
    base = s * CHUNK
    lane = lax.iota(jnp.int32, 16)
    zeros16 = jnp.zeros((16,), jnp.int32)
    ones16 = jnp.ones((16,), jnp.int32)

    pltpu.sync_copy(keys_hbm.at[pl.ds(base, CHUNK)], keys_v)

    prefix = jnp.uint32(0)
    gt_total = jnp.int32(0)
    for r in range(4):
        sh_dig = 24 - 8 * r

        def zero_step(i, c):
            hist_v[pl.ds(i * 16, 16)] = zeros16
            return c
        lax.fori_loop(0, 256, zero_step, 0, unroll=8)

        def hist_step(i, c):
            kv = keys_v[pl.ds(i * 16, 16)]
            dig = ((kv >> sh_dig) & jnp.uint32(0xFF)).astype(jnp.int32)
            slot = lane * 256 + dig
            if r == 0:
                m = jnp.full((16,), True)
            else:
                m = (kv >> (32 - 8 * r)) == prefix
            plsc.addupdate_scatter(hist_v, [slot], ones16, mask=m)
            return c
        lax.fori_loop(0, NV, hist_step, 0, unroll=4)

        def lred_step(c, _):
            acc = zeros16
            for l in range(NS):
                acc = acc + hist_v[pl.ds(l * 256 + c * 16, 16)]
            bins_v[pl.ds(c * 16, 16)] = acc
            return _
        lax.fori_loop(0, 16, lred_step, 0)

        pltpu.sync_copy(bins_v.at[pl.ds(0, 256)],
                        shist.at[pl.ds(s * 256, 256)])
        plsc.subcore_barrier()
        pltpu.sync_copy(shist, gbins_v)
        plsc.subcore_barrier()

        def gred_step(c, _):
            acc = zeros16
            for w in range(NS):
                acc = acc + gbins_v[pl.ds(w * 256 + c * 16, 16)]
            bins_v[pl.ds(c * 16, 16)] = acc
            return _
        lax.fori_loop(0, 16, gred_step, 0)

        k_rem = TOPK - gt_total

        def scan_step(t, carry):
            cum, bstar, gt_above = carry
            b = 255 - t
            cnt = bins_v[pl.ds(b, 16)][0]
            newcum = cum + cnt
            hit = (newcum >= k_rem) & (bstar < 0)
            return (newcum,
                    jnp.where(hit, b, bstar),
                    jnp.where(hit, cum, gt_above))
        _, bstar, gt_above = lax.fori_loop(
            0, 256, scan_step,
            (jnp.int32(0), jnp.int32(-1), jnp.int32(0)))
        gt_total = gt_total + gt_above
        prefix = (prefix << 8) | bstar.astype(jnp.uint32)

    thr = prefix
    n_gt = gt_total

    def comp_step(i, carry):
        cg, ce = carry
        kv = keys_v[pl.ds(i * 16, 16)]
        idxv = base + i * 16 + lane
        mgt = kv > thr
        meq = kv == thr
        plsc.store_compressed(gtk_v.at[pl.ds(cg, 16)], kv, mask=mgt)
        plsc.store_compressed(gti_v.at[pl.ds(cg, 16)], idxv, mask=mgt)
        ng = jnp.sum(mgt.astype(jnp.int32))
        ne = jnp.sum(meq.astype(jnp.int32))
        ok_eq = ce <= EQ_CAP - 16

        @pl.when(ok_eq)
        def _():
            plsc.store_compressed(eqi_v.at[pl.ds(ce, 16)], idxv, mask=meq)
        return (cg + ng, jnp.where(ok_eq, ce + ne, ce))
    cnt_gt, cnt_eq = lax.fori_loop(0, NV, comp_step,
                                   (jnp.int32(0), jnp.int32(0)))

    cnt_v[...] = jnp.where(lane == 0, cnt_gt,
                           jnp.where(lane == 1, cnt_eq, 0))
    pltpu.sync_copy(cnt_v, scnt.at[pl.ds(s * 16, 16)])
    plsc.subcore_barrier()
    pltpu.sync_copy(scnt, allcnt_v)
    plsc.subcore_barrier()

    def off_step(w, carry):
        go, eo, gtot, etot = carry
        row = allcnt_v[pl.ds(w * 16, 16)]
        cg = row[0]
        ce = row[1]
        before = w < s
        return (go + jnp.where(before, cg, 0),
                eo + jnp.where(before, ce, 0),
                gtot + cg, etot + ce)
    gt_off, eq_off, n_gt2, eq_tot = lax.fori_loop(
        0, NS, off_step,
        (jnp.int32(0), jnp.int32(0), jnp.int32(0), jnp.int32(0)))

    @pl.when(s == 0)
    def _():
        def pad_step(i, c):
            pad_k_v[pl.ds(i * 16, 16)] = jnp.zeros((16,), jnp.uint32)
            pad_i_v[pl.ds(i * 16, 16)] = jnp.full((16,), 0x7FFFFFFF,
                                                  jnp.int32)
            return c
        lax.fori_loop(0, SCAND // 16, pad_step, 0)
        pltpu.sync_copy(pad_k_v, scandk)
        pltpu.sync_copy(pad_i_v, scandi)
    plsc.subcore_barrier()

    dump = CAND + s * 8

    def posg_step(j, c):
        val = gt_off + j * 16 + lane
        ok = (j * 16 + lane) < cnt_gt
        posg_v[pl.ds(j * 16, 16)] = jnp.where(ok, val, dump)
        return c
    lax.fori_loop(0, GT_CAP // 16, posg_step, 0)

    def pose_step(j, c):
        val = TOPK + eq_off + j * 16 + lane
        ok = (j * 16 + lane) < cnt_eq
        pose_v[pl.ds(j * 16, 16)] = jnp.where(ok, val, dump)
        tk_v[pl.ds(j * 16, 16)] = jnp.full((16,), thr, jnp.uint32)
        return c
    lax.fori_loop(0, EQ_CAP // 16, pose_step, 0)

    pltpu.sync_copy(gtk_v, scandk.at[posg_v])
    pltpu.sync_copy(gti_v, scandi.at[posg_v])
    pltpu.sync_copy(tk_v, scandk.at[pose_v])
    pltpu.sync_copy(eqi_v, scandi.at[pose_v])
    plsc.subcore_barrier()

    pltpu.sync_copy(scandk.at[pl.ds(0, CAND)], ck_v.at[pl.ds(0, CAND)])
    pltpu.sync_copy(scandi.at[pl.ds(0, CAND)], ci_v.at[pl.ds(0, CAND)])

    nv_eq = (eq_tot + 15) // 16

    def slot_step(t, c):
        j = s + t * NS
        real = (j < n_gt2) | ((j >= TOPK) & (j < TOPK + eq_tot))
        tvec = jnp.full((16,), t, jnp.int32)
        m0 = lane == 0

        @pl.when(real)
        def _():
            kj = ck_v[pl.ds(j, 16)][0]
            ij = ci_v[pl.ds(j, 16)][0]

            def beat_count(v, acc):
                kv = ck_v[pl.ds(v * 16, 16)]
                iv = ci_v[pl.ds(v * 16, 16)]
                b = (kv > kj) | ((kv == kj) & (iv < ij))
                return acc + b.astype(jnp.int32)
            acc = lax.fori_loop(0, 64, beat_count, zeros16, unroll=8)
            acc = lax.fori_loop(64, 64 + nv_eq, beat_count, acc)
            rank = jnp.sum(acc)
            rk = jnp.where(rank < TOPK, rank, TOPK + s)
            plsc.store_scatter(rank_v, [tvec],
                               jnp.full((16,), rk, jnp.int32), mask=m0)
            plsc.store_scatter(gidx_v, [tvec],
                               jnp.full((16,), ij, jnp.int32), mask=m0)
            kj_i = lax.bitcast_convert_type(kj, jnp.int32)
            plsc.store_scatter(selk_v, [tvec],
                               jnp.full((16,), kj_i, jnp.int32), mask=m0)

        @pl.when(jnp.logical_not(real))
        def _():
            plsc.store_scatter(rank_v, [tvec],
                               jnp.full((16,), TOPK + s, jnp.int32), mask=m0)
            plsc.store_scatter(gidx_v, [tvec],
                               jnp.full((16,), j, jnp.int32), mask=m0)
            plsc.store_scatter(selk_v, [tvec], zeros16, mask=m0)
        return c
    lax.fori_loop(0, SLOTS, slot_step, 0)

    def val_step(t, c):
        u = lax.bitcast_convert_type(selk_v[pl.ds(t * 16, 16)], jnp.uint32)
        top = (u >> 31) == jnp.uint32(1)
        bits = jnp.where(top, u ^ jnp.uint32(0x80000000), ~u)
        vals_v[pl.ds(t * 16, 16)] = lax.bitcast_convert_type(bits,
                                                             jnp.float32)
        return c
    lax.fori_loop(0, SLOTS // 16, val_step, 0)

    pltpu.async_copy(x_hbm.at[gidx_v], rows_v, sem).wait()

    def scale_step(t, c):
        v = vals_v[pl.ds(t, 16)][0]
        for seg in range(D // 16):
            sl = pl.ds(seg * 16, 16)
            rows_v[t, sl] = rows_v[t, sl] * v
        return c
    lax.fori_loop(0, SLOTS, scale_step, 0, unroll=4)

    pltpu.sync_copy(rows_v, out_hbm.at[rank_v])


@jax.jit
def _pipeline(x, v):
    length = jnp.linalg.norm(v)
    keys = _make_keys(x, v, length.reshape(1, 1))

    mesh = plsc.VectorSubcoreMesh(core_axis_name="c", subcore_axis_name="s",
                                  num_cores=1)
    sc = pl.kernel(
        _sc_body,
        out_type=jax.ShapeDtypeStruct((OUT_PAD, D), jnp.float32),
        mesh=mesh,
        compiler_params=pltpu.CompilerParams(needs_layout_passes=False),
        scratch_types=[
            pltpu.VMEM((CHUNK,), jnp.uint32),
            pltpu.VMEM((4096,), jnp.int32),
            pltpu.VMEM((256 + 16,), jnp.int32),
            pltpu.VMEM((NS * 256,), jnp.int32),
            pltpu.VMEM((GT_CAP,), jnp.uint32),
            pltpu.VMEM((GT_CAP,), jnp.int32),
            pltpu.VMEM((EQ_CAP,), jnp.int32),
            pltpu.VMEM((EQ_CAP,), jnp.uint32),
            pltpu.VMEM((GT_CAP,), jnp.int32),
            pltpu.VMEM((EQ_CAP,), jnp.int32),
            pltpu.VMEM((16,), jnp.int32),
            pltpu.VMEM((NS * 16,), jnp.int32),
            pltpu.VMEM((SCAND,), jnp.uint32),
            pltpu.VMEM((SCAND,), jnp.int32),
            pltpu.VMEM((CAND + 16,), jnp.uint32),
            pltpu.VMEM((CAND + 16,), jnp.int32),
            pltpu.VMEM((SLOTS,), jnp.int32),
            pltpu.VMEM((SLOTS,), jnp.int32),
            pltpu.VMEM((SLOTS,), jnp.int32),
            pltpu.VMEM((SLOTS + 16,), jnp.float32),
            pltpu.VMEM((SLOTS, D), jnp.float32),
            pltpu.VMEM_SHARED((NS * 256,), jnp.int32),
            pltpu.VMEM_SHARED((NS * 16,), jnp.int32),
            pltpu.VMEM_SHARED((SCAND,), jnp.uint32),
            pltpu.VMEM_SHARED((SCAND,), jnp.int32),
            pltpu.SemaphoreType.DMA,
        ],
    )
    out_pad = sc(keys, x)
    return out_pad[:TOPK]


def kernel(x, learnable_vector):
    return _pipeline(x, learnable_vector)


# speedup vs baseline: 2.2172x; 1.3145x over previous
import functools

import jax
import jax.numpy as jnp
from jax import lax
from jax.experimental import pallas as pl
from jax.experimental.pallas import tpu as pltpu
from jax.experimental.pallas import tpu_sc as plsc

TOPK = 1024
N = 100000
D = 128
NS = 16
BLK_A = 2048
NPAD = 49 * BLK_A
CHUNK = NPAD // NS
NV = CHUNK // 16
GT_CAP = 1040
EQ_CAP = 96
CAND = 2048
SCAND = CAND + 128
SLOTS = CAND // NS
OUT_PAD = TOPK + NS


def _keys_body(x_ref, vt_ref, len_ref, k_ref):
    y = lax.dot_general(vt_ref[...], x_ref[...], (((1,), (1,)), ((), ())),
                        preferred_element_type=jnp.float32)
    y = y.reshape(BLK_A) / len_ref[0, 0]
    u = lax.bitcast_convert_type(y, jnp.uint32)
    sgn = lax.bitcast_convert_type(
        lax.shift_right_arithmetic(lax.bitcast_convert_type(y, jnp.int32), 31),
        jnp.uint32)
    k_ref[...] = u ^ (sgn | jnp.uint32(0x80000000))

    @pl.when(pl.program_id(0) == NPAD // BLK_A - 1)
    def _():
        k_ref[pl.ds(N - (NPAD - BLK_A), NPAD - N)] = jnp.zeros(
            (NPAD - N,), jnp.uint32)


def _make_keys(x, v, length):
    return pl.pallas_call(
        _keys_body,
        grid=(NPAD // BLK_A,),
        in_specs=[
            pl.BlockSpec((BLK_A, D), lambda i: (i, 0)),
            pl.BlockSpec((1, D), lambda i: (0, 0)),
            pl.BlockSpec((1, 1), lambda i: (0, 0)),
        ],
        out_specs=pl.BlockSpec((BLK_A,), lambda i: (i,)),
        out_shape=jax.ShapeDtypeStruct((NPAD,), jnp.uint32),
    )(x, v, length)


def _sc_body(keys_hbm, x_hbm, out_hbm,
             keys_v, hist_v, bins_v, gbins_v,
             gtk_v, gti_v, eqi_v, tk_v, posg_v, pose_v,
             cnt_v, allcnt_v, pad_k_v, pad_i_v,
             ck_v, ci_v, selk_v, rank_v, gidx_v, vals_v, rows_v,
             shist, scnt, scandk, scandi, sem):
    s = lax.axis_index("s")
    base = s * CHUNK
    lane = lax.iota(jnp.int32, 16)
    zeros16 = jnp.zeros((16,), jnp.int32)
    ones16 = jnp.ones((16,), jnp.int32)

    pltpu.sync_copy(keys_hbm.at[pl.ds(base, CHUNK)], keys_v)

    prefix = jnp.uint32(0)
    gt_total = jnp.int32(0)
    for r in range(4):
        sh_dig = 24 - 8 * r

        def zero_step(i, c):
            hist_v[pl.ds(i * 16, 16)] = zeros16
            return c
        lax.fori_loop(0, 256, zero_step, 0, unroll=8)

        def hist_step(i, c):
            kv = keys_v[pl.ds(i * 16, 16)]
            dig = ((kv >> sh_dig) & jnp.uint32(0xFF)).astype(jnp.int32)
            slot = lane * 256 + dig
            if r == 0:
                m = jnp.full((16,), True)
            else:
                m = (kv >> (32 - 8 * r)) == prefix
            plsc.addupdate_scatter(hist_v, [slot], ones16, mask=m)
            return c
        lax.fori_loop(0, NV, hist_step, 0, unroll=4)

        def lred_step(c, _):
            acc = zeros16
            for l in range(NS):
                acc = acc + hist_v[pl.ds(l * 256 + c * 16, 16)]
            bins_v[pl.ds(c * 16, 16)] = acc
            return _
        lax.fori_loop(0, 16, lred_step, 0)

        pltpu.sync_copy(bins_v.at[pl.ds(0, 256)],
                        shist.at[pl.ds(s * 256, 256)])
        plsc.subcore_barrier()
        pltpu.sync_copy(shist, gbins_v)
        plsc.subcore_barrier()

        def gred_step(c, _):
            acc = zeros16
            for w in range(NS):
                acc = acc + gbins_v[pl.ds(w * 256 + c * 16, 16)]
            bins_v[pl.ds(c * 16, 16)] = acc
            return _
        lax.fori_loop(0, 16, gred_step, 0)

        k_rem = TOPK - gt_total

        def scan_step(t, carry):
            cum, bstar, gt_above = carry
            b = 255 - t
            cnt = bins_v[pl.ds(b, 16)][0]
            newcum = cum + cnt
            hit = (newcum >= k_rem) & (bstar < 0)
            return (newcum,
                    jnp.where(hit, b, bstar),
                    jnp.where(hit, cum, gt_above))
        _, bstar, gt_above = lax.fori_loop(
            0, 256, scan_step,
            (jnp.int32(0), jnp.int32(-1), jnp.int32(0)))
        gt_total = gt_total + gt_above
        prefix = (prefix << 8) | bstar.astype(jnp.uint32)

    thr = prefix
    n_gt = gt_total

    def comp_step(i, carry):
        cg, ce = carry
        kv = keys_v[pl.ds(i * 16, 16)]
        idxv = base + i * 16 + lane
        mgt = kv > thr
        meq = kv == thr
        plsc.store_compressed(gtk_v.at[pl.ds(cg, 16)], kv, mask=mgt)
        plsc.store_compressed(gti_v.at[pl.ds(cg, 16)], idxv, mask=mgt)
        ng = jnp.sum(mgt.astype(jnp.int32))
        ne = jnp.sum(meq.astype(jnp.int32))
        ok_eq = ce <= EQ_CAP - 16

        @pl.when(ok_eq)
        def _():
            plsc.store_compressed(eqi_v.at[pl.ds(ce, 16)], idxv, mask=meq)
        return (cg + ng, jnp.where(ok_eq, ce + ne, ce))
    cnt_gt, cnt_eq = lax.fori_loop(0, NV, comp_step,
                                   (jnp.int32(0), jnp.int32(0)))

    cnt_v[...] = jnp.where(lane == 0, cnt_gt,
                           jnp.where(lane == 1, cnt_eq, 0))
    pltpu.sync_copy(cnt_v, scnt.at[pl.ds(s * 16, 16)])
    plsc.subcore_barrier()
    pltpu.sync_copy(scnt, allcnt_v)
    plsc.subcore_barrier()

    def off_step(w, carry):
        go, eo, gtot, etot = carry
        row = allcnt_v[pl.ds(w * 16, 16)]
        cg = row[0]
        ce = row[1]
        before = w < s
        return (go + jnp.where(before, cg, 0),
                eo + jnp.where(before, ce, 0),
                gtot + cg, etot + ce)
    gt_off, eq_off, n_gt2, eq_tot = lax.fori_loop(
        0, NS, off_step,
        (jnp.int32(0), jnp.int32(0), jnp.int32(0), jnp.int32(0)))

    @pl.when(s == 0)
    def _():
        def pad_step(i, c):
            pad_k_v[pl.ds(i * 16, 16)] = jnp.zeros((16,), jnp.uint32)
            pad_i_v[pl.ds(i * 16, 16)] = jnp.full((16,), 0x7FFFFFFF,
                                                  jnp.int32)
            return c
        lax.fori_loop(0, SCAND // 16, pad_step, 0)
        pltpu.sync_copy(pad_k_v, scandk)
        pltpu.sync_copy(pad_i_v, scandi)
    plsc.subcore_barrier()

    dump = CAND + s * 8

    def posg_step(j, c):
        val = gt_off + j * 16 + lane
        ok = (j * 16 + lane) < cnt_gt
        posg_v[pl.ds(j * 16, 16)] = jnp.where(ok, val, dump)
        return c
    lax.fori_loop(0, GT_CAP // 16, posg_step, 0)

    def pose_step(j, c):
        val = TOPK + eq_off + j * 16 + lane
        ok = (j * 16 + lane) < cnt_eq
        pose_v[pl.ds(j * 16, 16)] = jnp.where(ok, val, dump)
        tk_v[pl.ds(j * 16, 16)] = jnp.full((16,), thr, jnp.uint32)
        return c
    lax.fori_loop(0, EQ_CAP // 16, pose_step, 0)

    pltpu.sync_copy(gtk_v, scandk.at[posg_v])
    pltpu.sync_copy(gti_v, scandi.at[posg_v])
    pltpu.sync_copy(tk_v, scandk.at[pose_v])
    pltpu.sync_copy(eqi_v, scandi.at[pose_v])
    plsc.subcore_barrier()

    pltpu.sync_copy(scandk.at[pl.ds(0, CAND)], ck_v.at[pl.ds(0, CAND)])
    pltpu.sync_copy(scandi.at[pl.ds(0, CAND)], ci_v.at[pl.ds(0, CAND)])

    nv_eq = (eq_tot + 15) // 16

    def slot_step(t, c):
        j = s + t * NS
        real = (j < n_gt2) | ((j >= TOPK) & (j < TOPK + eq_tot))
        tvec = jnp.full((16,), t, jnp.int32)
        m0 = lane == 0

        @pl.when(real)
        def _():
            kj = ck_v[pl.ds(j, 16)][0]
            ij = ci_v[pl.ds(j, 16)][0]

            def beat_count(v, acc):
                kv = ck_v[pl.ds(v * 16, 16)]
                iv = ci_v[pl.ds(v * 16, 16)]
                b = (kv > kj) | ((kv == kj) & (iv < ij))
                return acc + b.astype(jnp.int32)
            acc = lax.fori_loop(0, 64, beat_count, zeros16, unroll=8)
            acc = lax.fori_loop(64, 64 + nv_eq, beat_count, acc)
            rank = jnp.sum(acc)
            rk = jnp.where(rank < TOPK, rank, TOPK + s)
            plsc.store_scatter(rank_v, [tvec],
                               jnp.full((16,), rk, jnp.int32), mask=m0)
            plsc.store_scatter(gidx_v, [tvec],
                               jnp.full((16,), ij, jnp.int32), mask=m0)
            kj_i = lax.bitcast_convert_type(kj, jnp.int32)
            plsc.store_scatter(selk_v, [tvec],
                               jnp.full((16,), kj_i, jnp.int32), mask=m0)

        @pl.when(jnp.logical_not(real))
        def _():
            plsc.store_scatter(rank_v, [tvec],
                               jnp.full((16,), TOPK + s, jnp.int32), mask=m0)
            plsc.store_scatter(gidx_v, [tvec],
                               jnp.full((16,), j, jnp.int32), mask=m0)
            plsc.store_scatter(selk_v, [tvec], zeros16, mask=m0)
        return c
    lax.fori_loop(0, SLOTS, slot_step, 0)

    def val_step(t, c):
        u = lax.bitcast_convert_type(selk_v[pl.ds(t * 16, 16)], jnp.uint32)
        top = (u >> 31) == jnp.uint32(1)
        bits = jnp.where(top, u ^ jnp.uint32(0x80000000), ~u)
        vals_v[pl.ds(t * 16, 16)] = lax.bitcast_convert_type(bits,
                                                             jnp.float32)
        return c
    lax.fori_loop(0, SLOTS // 16, val_step, 0)

    pltpu.async_copy(x_hbm.at[gidx_v], rows_v, sem).wait()

    def scale_step(t, c):
        v = vals_v[pl.ds(t, 16)][0]
        for seg in range(D // 16):
            sl = pl.ds(seg * 16, 16)
            rows_v[t, sl] = rows_v[t, sl] * v
        return c
    lax.fori_loop(0, SLOTS, scale_step, 0, unroll=4)

    pltpu.sync_copy(rows_v, out_hbm.at[rank_v])


@jax.jit
def _pipeline(x, v):
    length = jnp.linalg.norm(v)
    keys = _make_keys(x, v.T, length.reshape(1, 1))

    mesh = plsc.VectorSubcoreMesh(core_axis_name="c", subcore_axis_name="s",
                                  num_cores=1)
    sc = pl.kernel(
        _sc_body,
        out_type=jax.ShapeDtypeStruct((OUT_PAD, D), jnp.float32),
        mesh=mesh,
        compiler_params=pltpu.CompilerParams(needs_layout_passes=False),
        scratch_types=[
            pltpu.VMEM((CHUNK,), jnp.uint32),
            pltpu.VMEM((4096,), jnp.int32),
            pltpu.VMEM((256 + 16,), jnp.int32),
            pltpu.VMEM((NS * 256,), jnp.int32),
            pltpu.VMEM((GT_CAP,), jnp.uint32),
            pltpu.VMEM((GT_CAP,), jnp.int32),
            pltpu.VMEM((EQ_CAP,), jnp.int32),
            pltpu.VMEM((EQ_CAP,), jnp.uint32),
            pltpu.VMEM((GT_CAP,), jnp.int32),
            pltpu.VMEM((EQ_CAP,), jnp.int32),
            pltpu.VMEM((16,), jnp.int32),
            pltpu.VMEM((NS * 16,), jnp.int32),
            pltpu.VMEM((SCAND,), jnp.uint32),
            pltpu.VMEM((SCAND,), jnp.int32),
            pltpu.VMEM((CAND + 16,), jnp.uint32),
            pltpu.VMEM((CAND + 16,), jnp.int32),
            pltpu.VMEM((SLOTS,), jnp.int32),
            pltpu.VMEM((SLOTS,), jnp.int32),
            pltpu.VMEM((SLOTS,), jnp.int32),
            pltpu.VMEM((SLOTS + 16,), jnp.float32),
            pltpu.VMEM((SLOTS, D), jnp.float32),
            pltpu.VMEM_SHARED((NS * 256,), jnp.int32),
            pltpu.VMEM_SHARED((NS * 16,), jnp.int32),
            pltpu.VMEM_SHARED((SCAND,), jnp.uint32),
            pltpu.VMEM_SHARED((SCAND,), jnp.int32),
            pltpu.SemaphoreType.DMA,
        ],
    )
    out_pad = sc(keys, x)
    return out_pad[:TOPK]


def kernel(x, learnable_vector):
    return _pipeline(x, learnable_vector)


# speedup vs baseline: 2.7746x; 1.2514x over previous
import functools

import jax
import jax.numpy as jnp
from jax import lax
from jax.experimental import pallas as pl
from jax.experimental.pallas import tpu as pltpu
from jax.experimental.pallas import tpu_sc as plsc

TOPK = 1024
N = 100000
D = 128
NS = 16
BLK_A = 7168
NPAD = 14 * BLK_A
CHUNK = NPAD // NS
NV = CHUNK // 16
GT_CAP = 1040
EQ_CAP = 96
CAND = 2048
SCAND = CAND + 128
SLOTS = CAND // NS
OUT_PAD = TOPK + NS


def _keys_body(x_ref, vt_ref, len_ref, k_ref):
    y = lax.dot_general(vt_ref[...], x_ref[...], (((1,), (1,)), ((), ())),
                        preferred_element_type=jnp.float32)
    y = y.reshape(BLK_A) / len_ref[0, 0]
    u = lax.bitcast_convert_type(y, jnp.uint32)
    sgn = lax.bitcast_convert_type(
        lax.shift_right_arithmetic(lax.bitcast_convert_type(y, jnp.int32), 31),
        jnp.uint32)
    k_ref[...] = u ^ (sgn | jnp.uint32(0x80000000))

    @pl.when(pl.program_id(0) == NPAD // BLK_A - 1)
    def _():
        k_ref[pl.ds(N - (NPAD - BLK_A), NPAD - N)] = jnp.zeros(
            (NPAD - N,), jnp.uint32)


def _make_keys(x, v, length):
    return pl.pallas_call(
        _keys_body,
        grid=(NPAD // BLK_A,),
        in_specs=[
            pl.BlockSpec((BLK_A, D), lambda i: (i, 0)),
            pl.BlockSpec((1, D), lambda i: (0, 0)),
            pl.BlockSpec((1, 1), lambda i: (0, 0)),
        ],
        out_specs=pl.BlockSpec((BLK_A,), lambda i: (i,)),
        out_shape=jax.ShapeDtypeStruct((NPAD,), jnp.uint32),
    )(x, v, length)


def _sc_body(keys_hbm, x_hbm, out_hbm,
             keys_v, hist_v, bins_v, gbins_v,
             gtk_v, gti_v, eqi_v, tk_v, posg_v, pose_v,
             cnt_v, allcnt_v, pad_k_v, pad_i_v,
             ck_v, ci_v, selk_v, rank_v, gidx_v, vals_v, rows_v,
             shist, scnt, scandk, scandi, sem):
    s = lax.axis_index("s")
    base = s * CHUNK
    lane = lax.iota(jnp.int32, 16)
    zeros16 = jnp.zeros((16,), jnp.int32)
    ones16 = jnp.ones((16,), jnp.int32)

    pltpu.sync_copy(keys_hbm.at[pl.ds(base, CHUNK)], keys_v)

    prefix = jnp.uint32(0)
    gt_total = jnp.int32(0)
    for r in range(4):
        sh_dig = 24 - 8 * r

        def zero_step(i, c):
            hist_v[pl.ds(i * 16, 16)] = zeros16
            return c
        lax.fori_loop(0, 256, zero_step, 0, unroll=8)

        def hist_step(i, c):
            kv = keys_v[pl.ds(i * 16, 16)]
            dig = ((kv >> sh_dig) & jnp.uint32(0xFF)).astype(jnp.int32)
            slot = lane * 256 + dig
            if r == 0:
                m = jnp.full((16,), True)
            else:
                m = (kv >> (32 - 8 * r)) == prefix
            plsc.addupdate_scatter(hist_v, [slot], ones16, mask=m)
            return c
        lax.fori_loop(0, NV, hist_step, 0, unroll=4)

        def lred_step(c, _):
            acc = zeros16
            for l in range(NS):
                acc = acc + hist_v[pl.ds(l * 256 + c * 16, 16)]
            bins_v[pl.ds(c * 16, 16)] = acc
            return _
        lax.fori_loop(0, 16, lred_step, 0)

        pltpu.sync_copy(bins_v.at[pl.ds(0, 256)],
                        shist.at[pl.ds((r * NS + s) * 256, 256)])
        plsc.subcore_barrier()
        pltpu.sync_copy(shist.at[pl.ds(r * NS * 256, NS * 256)], gbins_v)

        def gred_step(c, carry):
            acc = zeros16
            for w in range(NS):
                acc = acc + gbins_v[pl.ds(w * 256 + c * 16, 16)]
            bins_v[pl.ds(c * 16, 16)] = acc
            return jnp.where(lane == c, jnp.sum(acc), carry)
        csums = lax.fori_loop(0, 16, gred_step, zeros16)

        k_rem = TOPK - gt_total
        suf_c = lax.rev(plsc.cumsum(lax.rev(csums, (0,))), (0,))
        cstar = jnp.sum((suf_c >= k_rem).astype(jnp.int32)) - 1
        base_above = jnp.sum(jnp.where(lane > cstar, csums, 0))
        bvec = bins_v[pl.ds(cstar * 16, 16)]
        suf_w = lax.rev(plsc.cumsum(lax.rev(bvec, (0,))), (0,))
        dstar = jnp.sum((base_above + suf_w >= k_rem).astype(jnp.int32)) - 1
        bstar = cstar * 16 + dstar
        gt_above = base_above + jnp.sum(jnp.where(lane > dstar, bvec, 0))
        gt_total = gt_total + gt_above
        prefix = (prefix << 8) | bstar.astype(jnp.uint32)

    thr = prefix
    n_gt = gt_total

    def comp_step(i, carry):
        cg, ce = carry
        kv = keys_v[pl.ds(i * 16, 16)]
        idxv = base + i * 16 + lane
        mgt = kv > thr
        meq = kv == thr
        plsc.store_compressed(gtk_v.at[pl.ds(cg, 16)], kv, mask=mgt)
        plsc.store_compressed(gti_v.at[pl.ds(cg, 16)], idxv, mask=mgt)
        ng = jnp.sum(mgt.astype(jnp.int32))
        ne = jnp.sum(meq.astype(jnp.int32))
        ok_eq = ce <= EQ_CAP - 16

        @pl.when(ok_eq)
        def _():
            plsc.store_compressed(eqi_v.at[pl.ds(ce, 16)], idxv, mask=meq)
        return (cg + ng, jnp.where(ok_eq, ce + ne, ce))
    cnt_gt, cnt_eq = lax.fori_loop(0, NV, comp_step,
                                   (jnp.int32(0), jnp.int32(0)))

    cnt_v[...] = jnp.where(lane == 0, cnt_gt,
                           jnp.where(lane == 1, cnt_eq, 0))
    pltpu.sync_copy(cnt_v, scnt.at[pl.ds(s * 16, 16)])

    @pl.when(s == 0)
    def _():
        def pad_step(i, c):
            pad_k_v[pl.ds(i * 16, 16)] = jnp.zeros((16,), jnp.uint32)
            pad_i_v[pl.ds(i * 16, 16)] = jnp.full((16,), 0x7FFFFFFF,
                                                  jnp.int32)
            return c
        lax.fori_loop(0, SCAND // 16, pad_step, 0, unroll=8)
        pltpu.sync_copy(pad_k_v, scandk)
        pltpu.sync_copy(pad_i_v, scandi)
    plsc.subcore_barrier()
    pltpu.sync_copy(scnt, allcnt_v)

    def off_step(w, carry):
        go, eo, gtot, etot = carry
        row = allcnt_v[pl.ds(w * 16, 16)]
        cg = row[0]
        ce = row[1]
        before = w < s
        return (go + jnp.where(before, cg, 0),
                eo + jnp.where(before, ce, 0),
                gtot + cg, etot + ce)
    gt_off, eq_off, n_gt2, eq_tot = lax.fori_loop(
        0, NS, off_step,
        (jnp.int32(0), jnp.int32(0), jnp.int32(0), jnp.int32(0)))

    dump = CAND + s * 8

    def posg_step(j, c):
        val = gt_off + j * 16 + lane
        ok = (j * 16 + lane) < cnt_gt
        posg_v[pl.ds(j * 16, 16)] = jnp.where(ok, val, dump)
        return c
    lax.fori_loop(0, GT_CAP // 16, posg_step, 0)

    def pose_step(j, c):
        val = TOPK + eq_off + j * 16 + lane
        ok = (j * 16 + lane) < cnt_eq
        pose_v[pl.ds(j * 16, 16)] = jnp.where(ok, val, dump)
        tk_v[pl.ds(j * 16, 16)] = jnp.full((16,), thr, jnp.uint32)
        return c
    lax.fori_loop(0, EQ_CAP // 16, pose_step, 0)

    pltpu.sync_copy(gtk_v, scandk.at[posg_v])
    pltpu.sync_copy(gti_v, scandi.at[posg_v])
    pltpu.sync_copy(tk_v, scandk.at[pose_v])
    pltpu.sync_copy(eqi_v, scandi.at[pose_v])
    plsc.subcore_barrier()

    pltpu.sync_copy(scandk.at[pl.ds(0, CAND)], ck_v.at[pl.ds(0, CAND)])
    pltpu.sync_copy(scandi.at[pl.ds(0, CAND)], ci_v.at[pl.ds(0, CAND)])

    nv_eq = (eq_tot + 15) // 16

    def slot_step(t, c):
        j = s + t * NS
        real = (j < n_gt2) | ((j >= TOPK) & (j < TOPK + eq_tot))
        tvec = jnp.full((16,), t, jnp.int32)
        m0 = lane == 0

        @pl.when(real)
        def _():
            kj = ck_v[pl.ds(j, 16)][0]
            ij = ci_v[pl.ds(j, 16)][0]

            def beat_count(v, acc):
                kv = ck_v[pl.ds(v * 16, 16)]
                iv = ci_v[pl.ds(v * 16, 16)]
                b = (kv > kj) | ((kv == kj) & (iv < ij))
                return acc + b.astype(jnp.int32)
            acc = lax.fori_loop(0, 64, beat_count, zeros16, unroll=8)
            acc = lax.fori_loop(64, 64 + nv_eq, beat_count, acc)
            rank = jnp.sum(acc)
            rk = jnp.where(rank < TOPK, rank, TOPK + s)
            plsc.store_scatter(rank_v, [tvec],
                               jnp.full((16,), rk, jnp.int32), mask=m0)
            plsc.store_scatter(gidx_v, [tvec],
                               jnp.full((16,), ij, jnp.int32), mask=m0)
            kj_i = lax.bitcast_convert_type(kj, jnp.int32)
            plsc.store_scatter(selk_v, [tvec],
                               jnp.full((16,), kj_i, jnp.int32), mask=m0)

        @pl.when(jnp.logical_not(real))
        def _():
            plsc.store_scatter(rank_v, [tvec],
                               jnp.full((16,), TOPK + s, jnp.int32), mask=m0)
            plsc.store_scatter(gidx_v, [tvec],
                               jnp.full((16,), j, jnp.int32), mask=m0)
            plsc.store_scatter(selk_v, [tvec], zeros16, mask=m0)
        return c
    lax.fori_loop(0, SLOTS, slot_step, 0)

    def val_step(t, c):
        u = lax.bitcast_convert_type(selk_v[pl.ds(t * 16, 16)], jnp.uint32)
        top = (u >> 31) == jnp.uint32(1)
        bits = jnp.where(top, u ^ jnp.uint32(0x80000000), ~u)
        vals_v[pl.ds(t * 16, 16)] = lax.bitcast_convert_type(bits,
                                                             jnp.float32)
        return c
    lax.fori_loop(0, SLOTS // 16, val_step, 0)

    pltpu.async_copy(x_hbm.at[gidx_v], rows_v, sem).wait()

    def scale_step(t, c):
        v = vals_v[pl.ds(t, 16)][0]
        for seg in range(D // 16):
            sl = pl.ds(seg * 16, 16)
            rows_v[t, sl] = rows_v[t, sl] * v
        return c
    lax.fori_loop(0, SLOTS, scale_step, 0, unroll=4)

    pltpu.sync_copy(rows_v, out_hbm.at[rank_v])


@jax.jit
def _pipeline(x, v):
    length = jnp.linalg.norm(v)
    keys = _make_keys(x, v.T, length.reshape(1, 1))

    mesh = plsc.VectorSubcoreMesh(core_axis_name="c", subcore_axis_name="s",
                                  num_cores=1)
    sc = pl.kernel(
        _sc_body,
        out_type=jax.ShapeDtypeStruct((OUT_PAD, D), jnp.float32),
        mesh=mesh,
        compiler_params=pltpu.CompilerParams(needs_layout_passes=False),
        scratch_types=[
            pltpu.VMEM((CHUNK,), jnp.uint32),
            pltpu.VMEM((4096,), jnp.int32),
            pltpu.VMEM((256 + 16,), jnp.int32),
            pltpu.VMEM((NS * 256,), jnp.int32),
            pltpu.VMEM((GT_CAP,), jnp.uint32),
            pltpu.VMEM((GT_CAP,), jnp.int32),
            pltpu.VMEM((EQ_CAP,), jnp.int32),
            pltpu.VMEM((EQ_CAP,), jnp.uint32),
            pltpu.VMEM((GT_CAP,), jnp.int32),
            pltpu.VMEM((EQ_CAP,), jnp.int32),
            pltpu.VMEM((16,), jnp.int32),
            pltpu.VMEM((NS * 16,), jnp.int32),
            pltpu.VMEM((SCAND,), jnp.uint32),
            pltpu.VMEM((SCAND,), jnp.int32),
            pltpu.VMEM((CAND + 16,), jnp.uint32),
            pltpu.VMEM((CAND + 16,), jnp.int32),
            pltpu.VMEM((SLOTS,), jnp.int32),
            pltpu.VMEM((SLOTS,), jnp.int32),
            pltpu.VMEM((SLOTS,), jnp.int32),
            pltpu.VMEM((SLOTS + 16,), jnp.float32),
            pltpu.VMEM((SLOTS, D), jnp.float32),
            pltpu.VMEM_SHARED((4 * NS * 256,), jnp.int32),
            pltpu.VMEM_SHARED((NS * 16,), jnp.int32),
            pltpu.VMEM_SHARED((SCAND,), jnp.uint32),
            pltpu.VMEM_SHARED((SCAND,), jnp.int32),
            pltpu.SemaphoreType.DMA,
        ],
    )
    out_pad = sc(keys, x)
    return out_pad[:TOPK]


def kernel(x, learnable_vector):
    return _pipeline(x, learnable_vector)
